# Initial kernel scaffold; baseline (speedup 1.0000x reference)
import jax
import jax.numpy as jnp
from jax.experimental import pallas as pl


def kernel(x, edge_index, W_in, b_in, W_conv0, b_conv0, gamma0, beta0, W_conv1, b_conv1, gamma1, beta1, W_conv2, b_conv2, gamma2, beta2, W_mlp1, b_mlp1, W_mlp2, b_mlp2, W_mlp3, b_mlp3):
    raise NotImplementedError("write your pallas kernel here")


# speedup vs baseline: 25.7724x; 25.7724x over previous
import functools

import jax
import jax.numpy as jnp
from jax import lax
from jax.experimental import pallas as pl
from jax.experimental.pallas import tpu as pltpu
from jax.experimental.pallas import tpu_sc as plsc

N = 100000
E = 1600000
H = 64

NC = 2
NS = 16
L = 16

N_PAD = 100352
E_ROWS = 12544
E_PAD = E_ROWS * 128
ROWS_PER_TILE = E_ROWS // NS
ROWS_PER_WORKER = E_ROWS // (NC * NS)
GRP = 4
NZ = N_PAD // NS
ZB = NZ // 16
BLK = 2048
GRID_N = N_PAD // BLK

@functools.cache
def _sc_mesh():
    return plsc.VectorSubcoreMesh(core_axis_name="c", subcore_axis_name="s",
                                  num_cores=NC, num_subcores=NS)


def _prep_body(src_ref, dst_ref, srcoff_ref, dstp_ref):
    b = pl.program_id(0)
    src = src_ref[...]
    dst = dst_ref[...]
    rows = jax.lax.broadcasted_iota(jnp.int32, (128, 128), 0) + b * 128
    lanes = jax.lax.broadcasted_iota(jnp.int32, (128, 128), 1)
    real = rows < (E // 128)
    pad_src = N + lanes
    pad_dst = N + (lanes % 8)
    dstp_ref[...] = jnp.where(real, dst, pad_dst)
    for c in range(4):
        srcoff_ref[c] = jnp.where(real, c * N_PAD + src, c * N_PAD + pad_src)


def _prep_edges(src2d, dst2d):
    nblk = E_ROWS // 128
    return pl.pallas_call(
        _prep_body,
        grid=(nblk,),
        in_specs=[
            pl.BlockSpec((128, 128), lambda b: (b, 0)),
            pl.BlockSpec((128, 128), lambda b: (b, 0)),
        ],
        out_specs=[
            pl.BlockSpec((4, 128, 128), lambda b: (0, b, 0)),
            pl.BlockSpec((128, 128), lambda b: (b, 0)),
        ],
        out_shape=[
            jax.ShapeDtypeStruct((4, E_ROWS, 128), jnp.int32),
            jax.ShapeDtypeStruct((E_ROWS, 128), jnp.int32),
        ],
    )(src2d, dst2d)


@functools.cache
def _sc_degree_kernel():
    return pl.kernel(
        _sc_degree_body,
        out_type=jax.ShapeDtypeStruct((NC, N_PAD), jnp.float32),
        mesh=_sc_mesh(),
        scratch_types=[
            pltpu.VMEM_SHARED((N_PAD,), jnp.float32),
            pltpu.VMEM((GRP, 128), jnp.int32),
            pltpu.VMEM((128,), jnp.float32),
            pltpu.VMEM((ZB,), jnp.float32),
        ],
        compiler_params=pltpu.CompilerParams(use_tc_tiling_on_sc=False),
    )


def _sc_degree_body(dstp_hbm, ones_hbm, zeros_hbm, out_hbm, acc, didx, ones_v,
                    zer_v):
    core = lax.axis_index("c")
    sub = lax.axis_index("s")
    pltpu.sync_copy(ones_hbm, ones_v)
    pltpu.sync_copy(zeros_hbm, zer_v)
    for z in range(16):
        pltpu.sync_copy(zer_v, acc.at[pl.ds(sub * NZ + z * ZB, ZB)])
    plsc.subcore_barrier()

    wid = core * NS + sub
    row0 = wid * ROWS_PER_WORKER

    def body(i, carry):
        base = row0 + i * GRP
        pltpu.sync_copy(dstp_hbm.at[pl.ds(base, GRP)], didx)
        for j in range(GRP):
            pltpu.sync_copy(ones_v, acc.at[didx.at[j]], add=True)
        return carry

    lax.fori_loop(0, ROWS_PER_WORKER // GRP, body, 0)
    plsc.subcore_barrier()
    pltpu.sync_copy(acc.at[pl.ds(sub * NZ, NZ)],
                    out_hbm.at[core].at[pl.ds(sub * NZ, NZ)])


RS = 4
BODY_ROWS = 16
NB = ROWS_PER_TILE // BODY_ROWS


@functools.cache
def _sc_conv_kernel():
    return pl.kernel(
        _sc_conv_body,
        out_type=jax.ShapeDtypeStruct((4, N_PAD, 16), jnp.float32),
        mesh=_sc_mesh(),
        scratch_types=[
            pltpu.VMEM_SHARED((N_PAD, 16), jnp.float32),
            pltpu.VMEM((2, 8, 128), jnp.int32),
            pltpu.VMEM((2, 8, 128), jnp.int32),
            pltpu.VMEM((2, RS * 128, 16), jnp.float32),
            pltpu.VMEM((ZB, 16), jnp.float32),
            pltpu.SemaphoreType.DMA((2,)),
            pltpu.SemaphoreType.DMA((4,)),
            pltpu.SemaphoreType.DMA((2,)),
            pltpu.SemaphoreType.DMA,
        ],
        compiler_params=pltpu.CompilerParams(use_tc_tiling_on_sc=False),
    )


def _sc_conv_body(g2_hbm, srcoff_hbm, dstp_hbm, zeros_hbm, out_hbm,
                  acc, sidx, didx, rows, zer_v, gsem, ssem, isem, zsem):
    core = lax.axis_index("c")
    sub = lax.axis_index("s")
    pltpu.sync_copy(zeros_hbm, zer_v)
    row0 = sub * ROWS_PER_TILE

    for half in range(2):
        for cidx in range(NC):
            chunk = 2 * half + cidx

            @pl.when(core == cidx)
            def _(chunk=chunk):
                zd = [pltpu.async_copy(zer_v,
                                       acc.at[pl.ds(sub * NZ + z * ZB, ZB)],
                                       zsem)
                      for z in range(NZ // ZB)]
                for d in zd:
                    d.wait()
                plsc.subcore_barrier()

                so = srcoff_hbm.at[chunk]

                def idx_descs(base, slot):
                    return [pltpu.make_async_copy(so.at[pl.ds(base, 8)],
                                                  sidx.at[slot],
                                                  isem.at[slot]),
                            pltpu.make_async_copy(dstp_hbm.at[pl.ds(base, 8)],
                                                  didx.at[slot],
                                                  isem.at[slot])]

                def g_descs(slot, jofs, ring):
                    return [pltpu.make_async_copy(
                        g2_hbm.at[sidx.at[slot].at[jofs + j]],
                        rows.at[ring].at[pl.ds(j * 128, 128)],
                        gsem.at[ring]) for j in range(RS)]

                def s_descs(slot, jofs, ring, phase):
                    return [pltpu.make_async_copy(
                        rows.at[ring].at[pl.ds(j * 128, 128)],
                        acc.at[didx.at[slot].at[jofs + j]],
                        ssem.at[phase]) for j in range(RS)]

                for d in idx_descs(row0, 0):
                    d.start()
                    d.wait()

                def body(k, carry):
                    base = row0 + k * BODY_ROWS
                    a_wait = idx_descs(row0, 0)
                    b = idx_descs(base + 8, 1)
                    g0 = g_descs(0, 0, 0)
                    g1 = g_descs(0, RS, 1)
                    g2d = g_descs(1, 0, 0)
                    g3 = g_descs(1, RS, 1)
                    s0 = s_descs(0, 0, 0, 0)
                    s1 = s_descs(0, RS, 1, 1)
                    s2 = s_descs(1, 0, 0, 2)
                    s3 = s_descs(1, RS, 1, 3)

                    @pl.when(k > 0)
                    def _():
                        for d in a_wait:
                            d.wait()
                        for d in s2 + s3:
                            d.wait()
                    for d in b:
                        d.start()
                    for d in g0 + g1:
                        d.start()
                    for gd, sd in zip(g0, s0):
                        gd.wait()
                        sd.start(add=True)
                    for gd, sd in zip(g1, s1):
                        gd.wait()
                        sd.start(add=True)
                    for d in b:
                        d.wait()
                    for d in s0:
                        d.wait()
                    for d in g2d:
                        d.start()
                    for d in s1:
                        d.wait()
                    for d in g3:
                        d.start()

                    @pl.when(k < NB - 1)
                    def _():
                        for d in idx_descs(base + BODY_ROWS, 0):
                            d.start()
                    for gd, sd in zip(g2d, s2):
                        gd.wait()
                        sd.start(add=True)
                    for gd, sd in zip(g3, s3):
                        gd.wait()
                        sd.start(add=True)
                    return carry

                lax.fori_loop(0, NB, body, 0)
                for d in (s_descs(1, 0, 0, 2) + s_descs(1, RS, 1, 3)):
                    d.wait()
                plsc.subcore_barrier()
                pltpu.sync_copy(
                    acc.at[pl.ds(sub * NZ, NZ)],
                    out_hbm.at[chunk].at[pl.ds(sub * NZ, NZ)])


def _dinv_body(d0_ref, d1_ref, e_ref, dpk_ref):
    deg = d0_ref[...] + d1_ref[...] + 1.0
    dinv8 = lax.rsqrt(deg)
    dpk_ref[...] = jnp.dot(dinv8, e_ref[...],
                           preferred_element_type=jnp.float32)


def _dinv_call(d0_8, d1_8, e8):
    return pl.pallas_call(
        _dinv_body,
        out_shape=jax.ShapeDtypeStruct((PR, 128), jnp.float32),
    )(d0_8, d1_8, e8)


def _row_mask(blk_idx, rows, cols):
    r = jax.lax.broadcasted_iota(jnp.int32, (rows, cols), 0) + blk_idx * rows
    return r < N


def _inproj_body(x2_ref, w_ref, b_ref, h_ref):
    b = pl.program_id(0)
    hv = jnp.dot(x2_ref[...], w_ref[...], preferred_element_type=jnp.float32)
    mask = _pk_row_mask(b)
    for c in range(4):
        hc = jnp.maximum(hv[:, 128 * c:128 * (c + 1)] + b_ref[c], 0.0)
        h_ref[c] = jnp.where(mask, hc, 0.0)


def _inproj(x2, w_big, b_pk):
    din = x2.shape[1]
    return pl.pallas_call(
        _inproj_body,
        grid=(GRID_N,),
        in_specs=[
            pl.BlockSpec((PBLK, din), lambda b: (b, 0)),
            pl.BlockSpec((din, 512), lambda b: (0, 0)),
            pl.BlockSpec((4, 1, 128), lambda b: (0, 0, 0)),
        ],
        out_specs=pl.BlockSpec((4, PBLK, 128), lambda b: (0, b, 0)),
        out_shape=jax.ShapeDtypeStruct((4, PR, 128), jnp.float32),
    )(x2, w_big, b_pk)


PR = N_PAD // 8
PBLK = BLK // 8


def _matscale_body(h_ref, w_ref, dinv_ref, g_ref):
    hcat = jnp.concatenate([h_ref[c] for c in range(4)], axis=1)
    gv = jnp.dot(hcat, w_ref[...], preferred_element_type=jnp.float32)
    dpk = dinv_ref[...]
    for c in range(4):
        g_ref[c] = gv[:, 128 * c:128 * (c + 1)] * dpk


def _matscale(h_pk, w_big, dinv_pk):
    return pl.pallas_call(
        _matscale_body,
        grid=(GRID_N,),
        in_specs=[
            pl.BlockSpec((4, PBLK, 128), lambda b: (0, b, 0)),
            pl.BlockSpec((512, 512), lambda b: (0, 0)),
            pl.BlockSpec((PBLK, 128), lambda b: (b, 0)),
        ],
        out_specs=pl.BlockSpec((4, PBLK, 128), lambda b: (0, b, 0)),
        out_shape=jax.ShapeDtypeStruct((4, PR, 128), jnp.float32),
    )(h_pk, w_big, dinv_pk)


def _pk_row_mask(blk_idx):
    r = jax.lax.broadcasted_iota(jnp.int32, (PBLK, 128), 0) + blk_idx * PBLK
    return r < (N // 8)


def _stats_body(outc_ref, g_ref, dpk_ref, bpk_ref, t_ref, s_ref, ss_ref):
    b = pl.program_id(0)

    @pl.when(b == 0)
    def _():
        s_ref[...] = jnp.zeros_like(s_ref)
        ss_ref[...] = jnp.zeros_like(ss_ref)

    mask = _pk_row_mask(b)
    dpk = dpk_ref[...]
    s_parts, ss_parts = [], []
    for c in range(4):
        t = dpk * (outc_ref[c] + g_ref[c]) + bpk_ref[c]
        t = jnp.where(mask, t, 0.0)
        t_ref[c] = t
        s_parts.append(jnp.sum(t, axis=0, keepdims=True))
        ss_parts.append(jnp.sum(t * t, axis=0, keepdims=True))
    s_ref[...] += jnp.stack(s_parts)
    ss_ref[...] += jnp.stack(ss_parts)


def _stats(out_pk, g_pk, dinv_pk, b_pk):
    return pl.pallas_call(
        _stats_body,
        grid=(GRID_N,),
        in_specs=[
            pl.BlockSpec((4, PBLK, 128), lambda b: (0, b, 0)),
            pl.BlockSpec((4, PBLK, 128), lambda b: (0, b, 0)),
            pl.BlockSpec((PBLK, 128), lambda b: (b, 0)),
            pl.BlockSpec((4, 1, 128), lambda b: (0, 0, 0)),
        ],
        out_specs=[
            pl.BlockSpec((4, PBLK, 128), lambda b: (0, b, 0)),
            pl.BlockSpec((4, 1, 128), lambda b: (0, 0, 0)),
            pl.BlockSpec((4, 1, 128), lambda b: (0, 0, 0)),
        ],
        out_shape=[
            jax.ShapeDtypeStruct((4, PR, 128), jnp.float32),
            jax.ShapeDtypeStruct((4, 1, 128), jnp.float32),
            jax.ShapeDtypeStruct((4, 1, 128), jnp.float32),
        ],
    )(out_pk, g_pk, dinv_pk, b_pk)


def _lane16_total(x):
    tot = jnp.sum(x.reshape(4, 8, 16), axis=1, keepdims=True)
    return jnp.broadcast_to(tot, (4, 8, 16)).reshape(4, 1, 128)


def _norm_body(t_ref, s_ref, ss_ref, gam_ref, bet_ref, hprev_ref, rfl_ref,
               h_ref):
    b = pl.program_id(0)
    mask = _pk_row_mask(b)
    mean = _lane16_total(s_ref[...]) / N
    var = _lane16_total(ss_ref[...]) / N - mean * mean
    inv = lax.rsqrt(var + 1e-5)
    rfl = rfl_ref[0, 0]
    for c in range(4):
        hn = (t_ref[c] - mean[c]) * inv[c] * gam_ref[c] + bet_ref[c]
        hn = jnp.where(mask, jnp.maximum(hn, 0.0), 0.0)
        h_ref[c] = hn + rfl * hprev_ref[c]


def _norm(t_pk, s, ss, gam_pk, bet_pk, h_prev_pk, rfl):
    return pl.pallas_call(
        _norm_body,
        grid=(GRID_N,),
        in_specs=[
            pl.BlockSpec((4, PBLK, 128), lambda b: (0, b, 0)),
            pl.BlockSpec((4, 1, 128), lambda b: (0, 0, 0)),
            pl.BlockSpec((4, 1, 128), lambda b: (0, 0, 0)),
            pl.BlockSpec((4, 1, 128), lambda b: (0, 0, 0)),
            pl.BlockSpec((4, 1, 128), lambda b: (0, 0, 0)),
            pl.BlockSpec((4, PBLK, 128), lambda b: (0, b, 0)),
            pl.BlockSpec((1, 1), lambda b: (0, 0)),
        ],
        out_specs=pl.BlockSpec((4, PBLK, 128), lambda b: (0, b, 0)),
        out_shape=jax.ShapeDtypeStruct((4, PR, 128), jnp.float32),
    )(t_pk, s, ss, gam_pk, bet_pk, h_prev_pk, rfl)


def _head_body(h_ref, w1_ref, b1_ref, w2_ref, b2_ref, w3_ref, b3_ref, o_ref):
    hcat = jnp.concatenate([h_ref[c] for c in range(4)], axis=1)
    o = jnp.dot(hcat, w1_ref[...], preferred_element_type=jnp.float32)
    o = jnp.maximum(o + b1_ref[...], 0.0)
    o = jnp.dot(o, w2_ref[...], preferred_element_type=jnp.float32)
    o = jnp.maximum(o + b2_ref[...], 0.0)
    o = jnp.dot(o, w3_ref[...], preferred_element_type=jnp.float32)
    o_ref[...] = jax.nn.sigmoid(o + b3_ref[...])


def _head(h_pk, w1, b1, w2, b2, w3, b3):
    return pl.pallas_call(
        _head_body,
        grid=(GRID_N,),
        in_specs=[
            pl.BlockSpec((4, PBLK, 128), lambda b: (0, b, 0)),
            pl.BlockSpec((512, 512), lambda b: (0, 0)),
            pl.BlockSpec((1, 512), lambda b: (0, 0)),
            pl.BlockSpec((512, 256), lambda b: (0, 0)),
            pl.BlockSpec((1, 256), lambda b: (0, 0)),
            pl.BlockSpec((256, 8), lambda b: (0, 0)),
            pl.BlockSpec((1, 8), lambda b: (0, 0)),
        ],
        out_specs=pl.BlockSpec((PBLK, 8), lambda b: (b, 0)),
        out_shape=jax.ShapeDtypeStruct((PR, 8), jnp.float32),
    )(h_pk, w1, b1, w2, b2, w3, b3)


def kernel(x, edge_index, W_in, b_in,
           W_conv0, b_conv0, gamma0, beta0,
           W_conv1, b_conv1, gamma1, beta1,
           W_conv2, b_conv2, gamma2, beta2,
           W_mlp1, b_mlp1, W_mlp2, b_mlp2, W_mlp3, b_mlp3):
    f32 = jnp.float32
    src2d = edge_index[0].reshape(E // 128, 128)
    dst2d = edge_index[1].reshape(E // 128, 128)
    src_off, dstp = _prep_edges(src2d, dst2d)

    ones128 = jnp.ones((128,), f32)
    zeros1 = jnp.zeros((ZB,), f32)
    zeros2 = jnp.zeros((ZB, 16), f32)
    degp = _sc_degree_kernel()(dstp, ones128, zeros1)
    eye8 = jnp.eye(8, dtype=f32)
    e8 = jnp.repeat(eye8, 16, axis=1)
    dinv_pk = _dinv_call(degp[0].reshape(PR, 8), degp[1].reshape(PR, 8), e8)

    def pk(v):
        return jnp.tile(v.reshape(4, 16), (1, 8)).reshape(4, 1, 128)

    def expand_conv(W):
        W4 = W.reshape(4, 16, 4, 16)
        return jnp.einsum('cidj,ab->caidbj', W4, eye8).reshape(512, 512)

    x2 = jnp.pad(x, ((0, N_PAD - N), (0, 3))).reshape(PR, 8 * 40)
    w_in_pad = jnp.pad(W_in, ((0, 3), (0, 0)))
    w_in_big = jnp.einsum('kdj,ab->akdbj', w_in_pad.reshape(40, 4, 16),
                          eye8).reshape(320, 512)
    h = _inproj(x2, w_in_big, pk(b_in))

    Ws = jnp.stack([expand_conv(W_conv0), expand_conv(W_conv1),
                    expand_conv(W_conv2)])
    bs = jnp.stack([pk(b_conv0), pk(b_conv1), pk(b_conv2)])
    gams = jnp.stack([pk(gamma0), pk(gamma1), pk(gamma2)])
    bets = jnp.stack([pk(beta0), pk(beta1), pk(beta2)])
    rfls = jnp.array([0.0, 1.0, 1.0], f32).reshape(3, 1, 1)

    def layer(h, xs):
        W_big, b_pk, gam_pk, bet_pk, rfl = xs
        g_pk = _matscale(h, W_big, dinv_pk)
        out_c = _sc_conv_kernel()(g_pk.reshape(4 * N_PAD, 16), src_off, dstp,
                                  zeros2)
        t_pk, st, ss = _stats(out_c.reshape(4, PR, 128), g_pk, dinv_pk, b_pk)
        h = _norm(t_pk, st, ss, gam_pk, bet_pk, h, rfl)
        return h, None

    h, _ = lax.scan(layer, h, (Ws, bs, gams, bets, rfls))

    w1_big = jnp.einsum('cij,ab->caibj', W_mlp1.reshape(4, 16, H),
                        eye8).reshape(512, 8 * H)
    w2_big = jnp.einsum('jm,ab->ajbm', W_mlp2, eye8).reshape(8 * H, 8 * 32)
    w3_big = jnp.einsum('m,ab->amb', W_mlp3[:, 0], eye8).reshape(8 * 32, 8)
    o = _head(h, w1_big, jnp.tile(b_mlp1, 8).reshape(1, 8 * H),
              w2_big, jnp.tile(b_mlp2, 8).reshape(1, 8 * 32),
              w3_big, jnp.tile(b_mlp3, 8).reshape(1, 8))
    return o.reshape(N_PAD)[:N]
